# Initial kernel scaffold; baseline (speedup 1.0000x reference)
import jax
import jax.numpy as jnp
from jax.experimental import pallas as pl


def kernel(x, ln_in_g, ln_in_b, W_in, b_in, conv_ln_g, conv_ln_b, edge_W, edge_b, self_W, conv_b, alpha, ln_out_g, ln_out_b, W_out, b_out):
    raise NotImplementedError("write your pallas kernel here")


# speedup vs baseline: 8.8068x; 8.8068x over previous
import functools

import jax
import jax.numpy as jnp
from jax import lax
from jax.experimental import pallas as pl
from jax.experimental.pallas import tpu as pltpu

C = 128
F_DIM = 256
H = 32
B = 4
L = 3
E = C * C - C

_INV_SQRT2 = 0.7071067811865476


def _gelu(v):
    return 0.5 * v * (1.0 + lax.erf(v * _INV_SQRT2))


def _ln(v, g, b, eps=1e-5):
    mu = jnp.mean(v, axis=-1, keepdims=True)
    var = jnp.mean((v - mu) ** 2, axis=-1, keepdims=True)
    return (v - mu) / jnp.sqrt(var + eps) * g + b


def _body(x_ref, ln_in_g_ref, ln_in_b_ref, w_in_ref, b_in_ref,
          conv_ln_g_ref, conv_ln_b_ref, edge_w_ref, edge_b_ref,
          self_w_ref, conv_b_ref, alpha_ref, ln_out_g_ref, ln_out_b_ref,
          w_out_ref, b_out_ref, out_ref, xbcm_ref, aggr_t_ref):
    f32 = jnp.float32
    xb = x_ref[0]

    xn_in = _ln(xb, ln_in_g_ref[...], ln_in_b_ref[...])
    h = lax.dot_general(xn_in, w_in_ref[...], (((1,), (0,)), ((), ())),
                        preferred_element_type=f32) + b_in_ref[...]
    h = _gelu(h)

    dot = lax.dot_general(h, h, (((1,), (1,)), ((), ())),
                          preferred_element_type=f32)
    rows = lax.broadcasted_iota(jnp.int32, (C, C), 0)
    cols = lax.broadcasted_iota(jnp.int32, (C, C), 1)
    eyef = (rows == cols).astype(f32)
    maskf = (rows != cols).astype(f32)
    n2c = jnp.sum(dot * eyef, axis=1, keepdims=True)
    n2r = jnp.sum(dot * eyef, axis=0, keepdims=True)
    nsc = jnp.maximum(jnp.sqrt(n2c), 1e-8)
    nsr = jnp.maximum(jnp.sqrt(n2r), 1e-8)
    cos = dot / (nsc * nsr)
    d2 = jnp.maximum(n2c + n2r - 2.0 * dot, 0.0)
    dist_raw = jnp.sqrt(d2)
    mean_dist = jnp.sum(dist_raw) * (1.0 / E)
    dist = dist_raw * (1.0 / (mean_dist + 1e-6))

    for l in range(L):
        xn = _ln(h, conv_ln_g_ref[l:l + 1, :], conv_ln_b_ref[l:l + 1, :])

        for hh in range(H):
            xcol = lax.slice(xn, (0, hh), (C, hh + 1))
            xbcm_ref[hh] = jnp.broadcast_to(xcol, (C, C)) * maskf

        def obody(o, carry, l=l):
            acc = jnp.zeros((C, C), f32)
            for hh in range(H):
                k = H * hh + o
                a = edge_w_ref[l, 0, k]
                bcf = edge_w_ref[l, 1, k]
                bb = edge_b_ref[l, k]
                t = cos * a + dist * bcf + bb
                acc = acc + jnp.tanh(t) * xbcm_ref[hh]
            aggr_t_ref[pl.ds(o, 1), :] = jnp.sum(acc, axis=0, keepdims=True)
            return carry

        lax.fori_loop(0, H, obody, 0)
        aggr = jnp.transpose(aggr_t_ref[...])

        self_term = lax.dot_general(xn, self_w_ref[l], (((1,), (0,)), ((), ())),
                                    preferred_element_type=f32)
        out_l = aggr * (1.0 / (C - 1)) + self_term + conv_b_ref[l:l + 1, :]
        h = h + alpha_ref[l] * out_l
        h = _gelu(h)

    hn = _ln(h, ln_out_g_ref[...], ln_out_b_ref[...])
    corr = lax.dot_general(hn, w_out_ref[...], (((1,), (0,)), ((), ())),
                           preferred_element_type=f32) + b_out_ref[...]
    out_ref[0] = xb + corr


@jax.jit
def kernel(x, ln_in_g, ln_in_b, W_in, b_in, conv_ln_g, conv_ln_b, edge_W,
           edge_b, self_W, conv_b, alpha, ln_out_g, ln_out_b, W_out, b_out):
    f32 = jnp.float32

    def vspec(shape):
        nd = len(shape)
        return pl.BlockSpec(shape, lambda b, nd=nd: (0,) * nd)

    def sspec():
        return pl.BlockSpec(memory_space=pltpu.SMEM)

    in_specs = [
        pl.BlockSpec((1, C, F_DIM), lambda b: (b, 0, 0)),
        vspec((1, F_DIM)),
        vspec((1, F_DIM)),
        vspec((F_DIM, H)),
        vspec((1, H)),
        vspec((L, H)),
        vspec((L, H)),
        sspec(),
        sspec(),
        vspec((L, H, H)),
        vspec((L, H)),
        sspec(),
        vspec((1, H)),
        vspec((1, H)),
        vspec((H, F_DIM)),
        vspec((1, F_DIM)),
    ]

    out = pl.pallas_call(
        _body,
        grid=(B,),
        in_specs=in_specs,
        out_specs=pl.BlockSpec((1, C, F_DIM), lambda b: (b, 0, 0)),
        out_shape=jax.ShapeDtypeStruct((B, C, F_DIM), f32),
        scratch_shapes=[
            pltpu.VMEM((H, C, C), f32),
            pltpu.VMEM((H, C), f32),
        ],
    )(
        x,
        ln_in_g.reshape(1, F_DIM), ln_in_b.reshape(1, F_DIM),
        W_in, b_in.reshape(1, H),
        conv_ln_g, conv_ln_b,
        edge_W, edge_b,
        self_W, conv_b,
        alpha,
        ln_out_g.reshape(1, H), ln_out_b.reshape(1, H),
        W_out, b_out.reshape(1, F_DIM),
    )
    return out
